# Initial kernel scaffold; baseline (speedup 1.0000x reference)
import functools

import jax
import jax.numpy as jnp
from jax.experimental import pallas as pl
from jax.experimental.pallas import tpu as pltpu
from jax.experimental.pallas import tpu_sc as plsc

VOCAB = 100001
N2 = 16
NF = 7
W = 128
K = 8
STEP = W * K


def _sc_gather(re_stack, im_stack, idx3, n):
    mesh = plsc.VectorSubcoreMesh(core_axis_name="c", subcore_axis_name="s")
    out = jax.ShapeDtypeStruct((n, NF * N2), jnp.float32)

    @functools.partial(pl.kernel, out_type=(out, out), mesh=mesh)
    def run(re_hbm, im_hbm, idx_hbm, ore_hbm, oim_hbm):
        def body(idx_v, ore_v, oim_v):
            for j in range(K):
                rows = idx_v.at[0, j]
                dst = pl.ds(j * W, W)
                pltpu.sync_copy(re_hbm.at[rows], ore_v.at[dst, :])
                pltpu.sync_copy(im_hbm.at[rows], oim_v.at[dst, :])

        pltpu.emit_pipeline(
            body,
            grid=(n // STEP, NF),
            in_specs=[pl.BlockSpec((1, K, W), lambda i, f: (f, i, 0))],
            out_specs=[
                pl.BlockSpec((STEP, N2), lambda i, f: (i, f)),
                pl.BlockSpec((STEP, N2), lambda i, f: (i, f)),
            ],
            core_axis_name=("c", "s"),
            dimension_semantics=(pltpu.PARALLEL, pltpu.PARALLEL),
        )(idx_hbm, ore_hbm, oim_hbm)

    return run(re_stack, im_stack, idx3)


def kernel(data, yr_re, yr_im, mt_re, mt_im, x_re, x_im, y_re, y_im,
           m_re, m_im, d_re, d_im, t_re, t_im):
    b, l, _ = data.shape
    n = b * l
    re_stack = jnp.concatenate([yr_re, mt_re, x_re, y_re, m_re, d_re, t_re],
                               axis=0)
    im_stack = jnp.concatenate([yr_im, mt_im, x_im, y_im, m_im, d_im, t_im],
                               axis=0)
    offs = jnp.arange(NF, dtype=jnp.int32) * VOCAB
    idx3 = (data.reshape(n, NF) + offs).T.reshape(NF, n // W, W)
    ore, oim = _sc_gather(re_stack, im_stack, idx3, n)
    return ore.reshape(b, l, NF * N2), oim.reshape(b, l, NF * N2)


# speedup vs baseline: 6.8300x; 6.8300x over previous
import functools

import jax
import jax.numpy as jnp
from jax.experimental import pallas as pl
from jax.experimental.pallas import tpu as pltpu
from jax.experimental.pallas import tpu_sc as plsc

VOCAB = 100001
N2 = 16
NF = 7
W = 128
K = 8
STEP = W * K


def _sc_gather(re_stack, im_stack, idx3, n):
    mesh = plsc.VectorSubcoreMesh(core_axis_name="c", subcore_axis_name="s")
    out = jax.ShapeDtypeStruct((n, NF * N2), jnp.float32)

    @functools.partial(
        pl.kernel, out_type=(out, out), mesh=mesh,
        compiler_params=pltpu.CompilerParams(use_tc_tiling_on_sc=False))
    def run(re_hbm, im_hbm, idx_hbm, ore_hbm, oim_hbm):
        def body(idx_v, ore_v, oim_v):
            for j in range(K):
                rows = idx_v.at[0, j]
                dst = pl.ds(j * W, W)
                pltpu.sync_copy(re_hbm.at[rows], ore_v.at[dst, :])
                pltpu.sync_copy(im_hbm.at[rows], oim_v.at[dst, :])

        pltpu.emit_pipeline(
            body,
            grid=(n // STEP, NF),
            in_specs=[pl.BlockSpec((1, K, W), lambda i, f: (f, i, 0))],
            out_specs=[
                pl.BlockSpec((STEP, N2), lambda i, f: (i, f)),
                pl.BlockSpec((STEP, N2), lambda i, f: (i, f)),
            ],
            core_axis_name=("c", "s"),
            dimension_semantics=(pltpu.PARALLEL, pltpu.PARALLEL),
        )(idx_hbm, ore_hbm, oim_hbm)

    return run(re_stack, im_stack, idx3)


def kernel(data, yr_re, yr_im, mt_re, mt_im, x_re, x_im, y_re, y_im,
           m_re, m_im, d_re, d_im, t_re, t_im):
    b, l, _ = data.shape
    n = b * l
    re_stack = jnp.concatenate([yr_re, mt_re, x_re, y_re, m_re, d_re, t_re],
                               axis=0)
    im_stack = jnp.concatenate([yr_im, mt_im, x_im, y_im, m_im, d_im, t_im],
                               axis=0)
    offs = jnp.arange(NF, dtype=jnp.int32) * VOCAB
    idx3 = (data.reshape(n, NF) + offs).T.reshape(NF, n // W, W)
    ore, oim = _sc_gather(re_stack, im_stack, idx3, n)
    return ore.reshape(b, l, NF * N2), oim.reshape(b, l, NF * N2)


# speedup vs baseline: 9.6973x; 1.4198x over previous
import functools

import jax
import jax.numpy as jnp
from jax.experimental import pallas as pl
from jax.experimental.pallas import tpu as pltpu
from jax.experimental.pallas import tpu_sc as plsc

VOCAB = 100001
N2 = 16
NF = 7
W = 128
K = 8
STEP = W * K


def _sc_gather(re_stack, im_stack, idx3, n):
    mesh = plsc.VectorSubcoreMesh(core_axis_name="c", subcore_axis_name="s")
    out = jax.ShapeDtypeStruct((n, NF * N2), jnp.float32)

    @functools.partial(
        pl.kernel, out_type=(out, out), mesh=mesh,
        scratch_types=[pltpu.SemaphoreType.DMA],
        compiler_params=pltpu.CompilerParams(use_tc_tiling_on_sc=False))
    def run(re_hbm, im_hbm, idx_hbm, ore_hbm, oim_hbm, sem):
        def body(idx_v, ore_v, oim_v):
            copies = []
            for j in range(K):
                rows = idx_v.at[0, j]
                dst = pl.ds(j * W, W)
                copies.append(
                    pltpu.async_copy(re_hbm.at[rows], ore_v.at[dst, :], sem))
                copies.append(
                    pltpu.async_copy(im_hbm.at[rows], oim_v.at[dst, :], sem))
            for c in copies:
                c.wait()

        pltpu.emit_pipeline(
            body,
            grid=(n // STEP, NF),
            in_specs=[pl.BlockSpec((1, K, W), lambda i, f: (f, i, 0))],
            out_specs=[
                pl.BlockSpec((STEP, N2), lambda i, f: (i, f)),
                pl.BlockSpec((STEP, N2), lambda i, f: (i, f)),
            ],
            core_axis_name=("c", "s"),
            dimension_semantics=(pltpu.PARALLEL, pltpu.PARALLEL),
        )(idx_hbm, ore_hbm, oim_hbm)

    return run(re_stack, im_stack, idx3)


def kernel(data, yr_re, yr_im, mt_re, mt_im, x_re, x_im, y_re, y_im,
           m_re, m_im, d_re, d_im, t_re, t_im):
    b, l, _ = data.shape
    n = b * l
    re_stack = jnp.concatenate([yr_re, mt_re, x_re, y_re, m_re, d_re, t_re],
                               axis=0)
    im_stack = jnp.concatenate([yr_im, mt_im, x_im, y_im, m_im, d_im, t_im],
                               axis=0)
    offs = jnp.arange(NF, dtype=jnp.int32) * VOCAB
    idx3 = (data.reshape(n, NF) + offs).T.reshape(NF, n // W, W)
    ore, oim = _sc_gather(re_stack, im_stack, idx3, n)
    return ore.reshape(b, l, NF * N2), oim.reshape(b, l, NF * N2)


# speedup vs baseline: 11.8454x; 1.2215x over previous
import functools

import jax
import jax.numpy as jnp
from jax import lax
from jax.experimental import pallas as pl
from jax.experimental.pallas import tpu as pltpu
from jax.experimental.pallas import tpu_sc as plsc

N2 = 16
NF = 7
W = 128
K = 8
STEP = K * W
LANES = 16


def _sc_gather(tabs, idx2, n):
    mesh = plsc.VectorSubcoreMesh(core_axis_name="c", subcore_axis_name="s")
    out = jax.ShapeDtypeStruct((n, NF * N2), jnp.float32)

    @functools.partial(
        pl.kernel, out_type=(out, out), mesh=mesh,
        scratch_types=[pltpu.VMEM((K, W), jnp.int32),
                       pltpu.SemaphoreType.DMA],
        compiler_params=pltpu.CompilerParams(use_tc_tiling_on_sc=False,
                                             needs_layout_passes=False))
    def run(*refs):
        tab_refs = refs[:2 * NF]
        idx_hbm, ore_hbm, oim_hbm, idx_s, sem = refs[2 * NF:]
        re_refs = tab_refs[0::2]
        im_refs = tab_refs[1::2]

        def body(idxs, idx_v, ore_v, oim_v):
            _, f = idxs
            lanes = lax.iota(jnp.int32, 16)
            cols = jnp.full((LANES,), f, jnp.int32)
            for v in range(STEP // LANES):
                rows = lanes + (LANES * v)
                vals = plsc.load_gather(idx_v, [rows, cols])
                idx_s[v // (W // LANES),
                      pl.ds((v % (W // LANES)) * LANES, LANES)] = vals
            for ff in range(NF):
                @pl.when(f == ff)
                def _(ff=ff):
                    for k in range(K):
                        rows_ref = idx_s.at[k]
                        dst = pl.ds(k * W, W)
                        pltpu.async_copy(re_refs[ff].at[rows_ref],
                                         ore_v.at[dst, :], sem)
                        pltpu.async_copy(im_refs[ff].at[rows_ref],
                                         oim_v.at[dst, :], sem)
            for k in range(K):
                rows_ref = idx_s.at[k]
                dst = pl.ds(k * W, W)
                pltpu.make_async_copy(re_refs[0].at[rows_ref],
                                      ore_v.at[dst, :], sem).wait()
                pltpu.make_async_copy(im_refs[0].at[rows_ref],
                                      oim_v.at[dst, :], sem).wait()

        pltpu.emit_pipeline(
            body,
            grid=(n // STEP, NF),
            in_specs=[pl.BlockSpec((STEP, NF), lambda i, f: (i, 0))],
            out_specs=[
                pl.BlockSpec((STEP, N2), lambda i, f: (i, f)),
                pl.BlockSpec((STEP, N2), lambda i, f: (i, f)),
            ],
            core_axis_name=("c", "s"),
            dimension_semantics=(pltpu.PARALLEL, pltpu.PARALLEL),
            _explicit_indices=True,
        )(idx_hbm, ore_hbm, oim_hbm)

    return run(*tabs, idx2)


def kernel(data, yr_re, yr_im, mt_re, mt_im, x_re, x_im, y_re, y_im,
           m_re, m_im, d_re, d_im, t_re, t_im):
    b, l, _ = data.shape
    n = b * l
    tabs = (yr_re, yr_im, mt_re, mt_im, x_re, x_im, y_re, y_im,
            m_re, m_im, d_re, d_im, t_re, t_im)
    ore, oim = _sc_gather(tabs, data.reshape(n, NF), n)
    return ore.reshape(b, l, NF * N2), oim.reshape(b, l, NF * N2)


# speedup vs baseline: 14.5316x; 1.2268x over previous
import functools

import jax
import jax.numpy as jnp
from jax.experimental import pallas as pl
from jax.experimental.pallas import tpu as pltpu
from jax.experimental.pallas import tpu_sc as plsc

N2 = 16
NF = 7
W = 128
K = 8


def _sc_gather(tabs, idx3, b, l):
    mesh = plsc.VectorSubcoreMesh(core_axis_name="c", subcore_axis_name="s")
    out = jax.ShapeDtypeStruct((b, l, NF * N2), jnp.float32)
    bblocks = b // (K * W)

    @functools.partial(
        pl.kernel, out_type=(out, out), mesh=mesh,
        scratch_types=[pltpu.SemaphoreType.DMA],
        compiler_params=pltpu.CompilerParams(use_tc_tiling_on_sc=False,
                                             needs_layout_passes=False))
    def run(*refs):
        tab_refs = refs[:2 * NF]
        idx_hbm, ore_hbm, oim_hbm, sem = refs[2 * NF:]
        re_refs = tab_refs[0::2]
        im_refs = tab_refs[1::2]

        def body(idxs, idx_v, ore_v, oim_v):
            _, f = idxs
            for ff in range(NF):
                @pl.when(f == ff)
                def _(ff=ff):
                    for k in range(K):
                        rows_ref = idx_v.at[0, k]
                        dst = pl.ds(k * W, W)
                        pltpu.async_copy(re_refs[ff].at[rows_ref],
                                         ore_v.at[dst, 0, :], sem)
                        pltpu.async_copy(im_refs[ff].at[rows_ref],
                                         oim_v.at[dst, 0, :], sem)
            for k in range(K):
                rows_ref = idx_v.at[0, k]
                dst = pl.ds(k * W, W)
                pltpu.make_async_copy(re_refs[0].at[rows_ref],
                                      ore_v.at[dst, 0, :], sem).wait()
                pltpu.make_async_copy(im_refs[0].at[rows_ref],
                                      oim_v.at[dst, 0, :], sem).wait()

        out_spec = pl.BlockSpec(
            (K * W, 1, N2),
            lambda i, f: (i % bblocks, i // bblocks, f))
        pltpu.emit_pipeline(
            body,
            grid=(b * l // (K * W), NF),
            in_specs=[pl.BlockSpec((1, K, W), lambda i, f: (f, i, 0))],
            out_specs=[out_spec, out_spec],
            core_axis_name=("c", "s"),
            dimension_semantics=(pltpu.PARALLEL, pltpu.PARALLEL),
            _explicit_indices=True,
        )(idx_hbm, ore_hbm, oim_hbm)

    return run(*tabs, idx3)


def kernel(data, yr_re, yr_im, mt_re, mt_im, x_re, x_im, y_re, y_im,
           m_re, m_im, d_re, d_im, t_re, t_im):
    b, l, _ = data.shape
    tabs = (yr_re, yr_im, mt_re, mt_im, x_re, x_im, y_re, y_im,
            m_re, m_im, d_re, d_im, t_re, t_im)
    idx3 = jnp.transpose(data, (2, 1, 0)).reshape(NF, b * l // W, W)
    return _sc_gather(tabs, idx3, b, l)
